# baseline (speedup 1.0000x reference)
import jax
import jax.numpy as jnp
from jax import lax
from jax.experimental import pallas as pl
from jax.experimental.pallas import tpu as pltpu
from jax.experimental.pallas import tpu_sc as plsc

N = 4096
E = 131072
IN = 128
HID = 64
HEADS = 4
D = HID * HEADS
T = 29

NW = 32
EPW = E // NW
EPT = E // 16
BR = 128
NB = N // 2 // BR
DEADCOL = N
DEN = 4 * N + 2048
NSLOT = 32


def _mesh():
    return plsc.VectorSubcoreMesh(core_axis_name="c", subcore_axis_name="s")


_params = pltpu.CompilerParams(needs_layout_passes=False)


def _iota16():
    return lax.iota(jnp.int32, 16)


def _splat_i(x):
    return jnp.full((16,), x, jnp.int32)


def _feat_kernel(h_ref, wt_ref, wg_ref, al_ref, ar_ref, feat_ref, el_ref, ert_ref):
    hw = lax.dot_general(h_ref[...], wt_ref[...], (((1,), (1,)), ((), ())),
                         precision=lax.Precision.HIGHEST)
    hh = jnp.maximum(hw, 0.01 * hw)
    feat = lax.dot_general(hh, wg_ref[...], (((1,), (1,)), ((), ())),
                           precision=lax.Precision.HIGHEST)
    feat_ref[...] = feat
    fh = feat.reshape(N, HEADS, HID)
    el_ref[...] = jnp.sum(fh * al_ref[...][None], axis=-1)
    ert_ref[...] = jnp.sum(fh * ar_ref[...][None], axis=-1).T


def _final_kernel(rst_ref, b_ref, out_ref):
    x = rst_ref[...] + b_ref[...]
    out_ref[...] = jnp.where(x > 0, x, jnp.exp(jnp.minimum(x, 0.0)) - 1.0)


CH = 128


def _sval_body(src_hbm, dst_hbm, sim_hbm, out_hbm, rowi, lanes, svalv,
               rowbuf0, rowbuf1, sem):
    c = lax.axis_index("c")
    s = lax.axis_index("s")
    wid = s * 2 + c
    base = wid * EPW
    pltpu.sync_copy(src_hbm.at[pl.ds(base, EPW)], rowi)
    pltpu.sync_copy(dst_hbm.at[pl.ds(base, EPW)], lanes)

    def fill(i, _):
        sv = rowi[pl.ds(i * 16, 16)]
        dv = lanes[pl.ds(i * 16, 16)]
        rowi[pl.ds(i * 16, 16)] = sv * 32 + lax.shift_right_logical(dv, 7)
        lanes[pl.ds(i * 16, 16)] = lax.bitwise_and(dv, 127)
        return 0

    lax.fori_loop(0, EPW // 16, fill, 0, unroll=4)

    nch = EPW // CH
    bufs = (rowbuf0, rowbuf1)
    started = [
        pltpu.async_copy(sim_hbm.at[rowi.at[pl.ds(j * CH, CH)]], bufs[j], sem)
        for j in range(2)
    ]

    for j in range(nch):
        started[j].wait()

        def extract(i, _):
            p = _iota16() + i * 16
            lane = lanes[pl.ds(j * CH + i * 16, 16)]
            svalv[pl.ds(j * CH + i * 16, 16)] = plsc.load_gather(
                bufs[j % 2], [p, lane])
            return 0

        lax.fori_loop(0, CH // 16, extract, 0, unroll=4)
        if j + 2 < nch:
            started.append(
                pltpu.async_copy(
                    sim_hbm.at[rowi.at[pl.ds((j + 2) * CH, CH)]],
                    bufs[j % 2], sem))

    pltpu.sync_copy(svalv, out_hbm.at[pl.ds(base, EPW)])


_sval_call = pl.kernel(
    _sval_body,
    out_type=jax.ShapeDtypeStruct((E,), jnp.float32),
    mesh=_mesh(),
    compiler_params=_params,
    scratch_types=[
        pltpu.VMEM((EPW,), jnp.int32),
        pltpu.VMEM((EPW,), jnp.int32),
        pltpu.VMEM((EPW,), jnp.float32),
        pltpu.VMEM((CH, 128), jnp.float32),
        pltpu.VMEM((CH, 128), jnp.float32),
        pltpu.SemaphoreType.DMA,
    ],
)


def _graph_body(ef_hbm, sv_hbm, el_hbm, ert_hbm,
                kdst_hbm, kex_hbm, den_hbm,
                eflat, svv, fidx, fval, tmp128, tmp128m, erbuf, elbuf,
                rowbuf, markbuf, gbuf, onesb, kcol, ccol, kdst_st, kex_st,
                denidx, denval, zbuf, acc, den_sh, mark_sh):
    c = lax.axis_index("c")
    t = lax.axis_index("s")

    ebase = t * EPT
    pltpu.sync_copy(ef_hbm.at[pl.ds(ebase, EPT)], eflat)
    pltpu.sync_copy(sv_hbm.at[pl.ds(ebase, EPT)], svv)
    pltpu.sync_copy(ert_hbm, erbuf)

    def zb(i, _):
        zbuf[pl.ds(i * 16, 16)] = jnp.zeros((16,), jnp.float32)
        return 0
    lax.fori_loop(0, 8192 // 16, zb, 0, unroll=8)

    def ob(i, _):
        onesb[pl.ds(i * 16, 16)] = jnp.ones((16,), jnp.float32)
        return 0
    lax.fori_loop(0, 128 // 16, ob, 0)

    @pl.when(t == 0)
    def _():
        pltpu.sync_copy(zbuf, den_sh.at[pl.ds(0, 8192)])
        pltpu.sync_copy(zbuf, den_sh.at[pl.ds(8192, 8192)])
        pltpu.sync_copy(zbuf.at[pl.ds(0, 2048)], den_sh.at[pl.ds(16384, 2048)])

    def block_body(b, _):
        blo = c * (N // 2) + b * BR

        plsc.subcore_barrier()
        for k in range(4):
            pltpu.sync_copy(zbuf, acc.at[pl.ds(t * (BR * N // 16) + k * 8192,
                                               8192)])
        pltpu.sync_copy(zbuf.at[pl.ds(0, 2048)],
                        mark_sh.at[pl.ds(t * 2048, 2048)])

        @pl.when(t == 0)
        def _():
            pltpu.sync_copy(zbuf.at[pl.ds(0, 2048)],
                            mark_sh.at[pl.ds(BR * 256, 2048)])
        plsc.subcore_barrier()

        def scan(i, cnt):
            ef = eflat[pl.ds(i * 16, 16)]
            sv = lax.shift_right_logical(ef, 12)
            vv = svv[pl.ds(i * 16, 16)]
            inb = (sv >= blo) & (sv < blo + BR)
            lidx = ef - blo * N
            plsc.store_compressed(fidx.at[pl.ds(cnt, 16)], lidx, mask=inb)
            plsc.store_compressed(fval.at[pl.ds(cnt, 16)], vv, mask=inb)
            n = jnp.max(plsc.all_reduce_population_count(inb))
            return cnt + n

        cnt = lax.fori_loop(0, EPT // 16, scan, 0)
        for k in range(8):
            fidx[pl.ds(cnt + k * 16, 16)] = _splat_i(BR * N)
            fval[pl.ds(cnt + k * 16, 16)] = jnp.zeros((16,), jnp.float32)
        nst = lax.div(cnt + 127, 128)

        def stream_j(j, _):
            for k in range(8):
                fv = fidx[pl.ds(j * 128 + k * 16, 16)]
                tmp128[pl.ds(k * 16, 16)] = fv
                tmp128m[pl.ds(k * 16, 16)] = lax.shift_right_logical(fv, 4)
            pltpu.sync_copy(fval.at[pl.ds(j * 128, 128)], acc.at[tmp128],
                            add=True)
            pltpu.sync_copy(onesb, mark_sh.at[tmp128m], add=True)
            return 0

        lax.fori_loop(0, nst, stream_j, 0)
        plsc.subcore_barrier()

        rpt = BR // 16
        r0 = blo + t * rpt
        pltpu.sync_copy(el_hbm.at[pl.ds(r0 * 4, rpt * 4)], elbuf)

        def row_body(rr, _):
            pltpu.sync_copy(acc.at[pl.ds((t * rpt + rr) * N, N)], rowbuf)
            pltpu.sync_copy(mark_sh.at[pl.ds((t * rpt + rr) * 256, 256)],
                            markbuf)

            def mscan(j, cur):
                m = markbuf[pl.ds(j * 16, 16)] > 0.0
                plsc.store_compressed(gbuf.at[pl.ds(cur, 16)],
                                      _iota16() + j * 16, mask=m)
                return cur + jnp.max(plsc.all_reduce_population_count(m))

            ngrp = lax.fori_loop(0, 16, mscan, 0)

            def comp(gi, cur):
                g = jnp.max(plsc.load_gather(gbuf, [_splat_i(gi)]))
                v = rowbuf[pl.ds(g * 16, 16)]
                m = v > 0.0
                plsc.store_compressed(ccol.at[pl.ds(cur, 16)],
                                      _iota16() + g * 16, mask=m)
                return cur + jnp.max(plsc.all_reduce_population_count(m))

            P = lax.fori_loop(0, ngrp, comp, 0)
            kc = jnp.minimum(P, T)

            def take_all(_):
                kcol[pl.ds(0, 16)] = ccol[pl.ds(0, 16)]
                kcol[pl.ds(16, 16)] = ccol[pl.ds(16, 16)]
                return 0

            def select29(_):
                def pick(k, _):
                    def mx_scan(j, mx):
                        return jnp.maximum(mx, rowbuf[pl.ds(j * 16, 16)])

                    mxv = lax.fori_loop(0, N // 16, mx_scan,
                                        jnp.full((16,), -jnp.inf, jnp.float32))
                    mxs = jnp.max(mxv)

                    def find(j, pos):
                        v = rowbuf[pl.ds(j * 16, 16)]
                        m = v == mxs
                        anyc = jnp.max(
                            plsc.all_reduce_population_count(m)) > 0
                        ffs = jnp.max(plsc.all_reduce_ffs(m))
                        return jnp.where((pos < 0) & anyc, j * 16 + ffs, pos)

                    pos = lax.fori_loop(0, N // 16, find, jnp.int32(-1))
                    plsc.store_scatter(kcol, [_splat_i(k)], _splat_i(pos),
                                       mask=_iota16() == 0)
                    plsc.store_scatter(rowbuf, [_splat_i(pos)],
                                       jnp.full((16,), -jnp.inf, jnp.float32),
                                       mask=_iota16() == 0)
                    return 0

                lax.fori_loop(0, T, pick, 0)
                return 0

            lax.cond(P <= T, take_all, select29, 0)

            for half in range(2):
                colv = kcol[pl.ds(half * 16, 16)]
                vmask = (_iota16() + half * 16) < kc
                colk = jnp.where(vmask, colv, DEADCOL)
                col_er = jnp.where(vmask, colv, 0)
                kdst_st[pl.ds(rr * NSLOT + half * 16, 16)] = colk
                for hh in range(HEADS):
                    erh = plsc.load_gather(erbuf, [_splat_i(hh), col_er])
                    els = plsc.load_gather(elbuf, [_splat_i(rr * 4 + hh)])
                    e = els + erh
                    e = jnp.where(e > 0, e, 0.2 * e)
                    ex = jnp.where(vmask, jnp.exp(e), 0.0)
                    off = rr * 128 + hh * 32 + half * 16
                    kex_st[pl.ds(off, 16)] = ex
                    denidx[pl.ds(off, 16)] = colk * 4 + hh
                    denval[pl.ds(off, 16)] = ex
            return 0

        lax.fori_loop(0, rpt, row_body, 0)

        def den_j(j, _):
            for k in range(8):
                tmp128[pl.ds(k * 16, 16)] = denidx[pl.ds(j * 128 + k * 16, 16)]
            pltpu.sync_copy(denval.at[pl.ds(j * 128, 128)], den_sh.at[tmp128],
                            add=True)
            return 0

        lax.fori_loop(0, rpt, den_j, 0)

        pltpu.sync_copy(kdst_st, kdst_hbm.at[pl.ds(r0 * NSLOT, rpt * NSLOT)])
        pltpu.sync_copy(kex_st, kex_hbm.at[pl.ds(r0 * 128, rpt * 128)])
        return 0

    lax.fori_loop(0, NB, block_body, 0)
    plsc.subcore_barrier()

    @pl.when(t == 0)
    def _():
        pltpu.sync_copy(den_sh, den_hbm.at[pl.ds(c * DEN, DEN)])


_graph_call = pl.kernel(
    _graph_body,
    out_type=(
        jax.ShapeDtypeStruct((N * NSLOT,), jnp.int32),
        jax.ShapeDtypeStruct((N * 128,), jnp.float32),
        jax.ShapeDtypeStruct((2 * DEN,), jnp.float32),
    ),
    mesh=_mesh(),
    compiler_params=_params,
    scratch_types=[
        pltpu.VMEM((EPT,), jnp.int32),
        pltpu.VMEM((EPT,), jnp.float32),
        pltpu.VMEM((EPT + 256,), jnp.int32),
        pltpu.VMEM((EPT + 256,), jnp.float32),
        pltpu.VMEM((128,), jnp.int32),
        pltpu.VMEM((128,), jnp.int32),
        pltpu.VMEM((HEADS, N), jnp.float32),
        pltpu.VMEM((32,), jnp.float32),
        pltpu.VMEM((N,), jnp.float32),
        pltpu.VMEM((256,), jnp.float32),
        pltpu.VMEM((272,), jnp.int32),
        pltpu.VMEM((128,), jnp.float32),
        pltpu.VMEM((48,), jnp.int32),
        pltpu.VMEM((N + 16,), jnp.int32),
        pltpu.VMEM((BR // 16 * NSLOT,), jnp.int32),
        pltpu.VMEM((BR // 16 * 128,), jnp.float32),
        pltpu.VMEM((BR // 16 * 128,), jnp.int32),
        pltpu.VMEM((BR // 16 * 128,), jnp.float32),
        pltpu.VMEM((8192,), jnp.float32),
        pltpu.VMEM_SHARED((BR * N + 2048,), jnp.float32),
        pltpu.VMEM_SHARED((DEN,), jnp.float32),
        pltpu.VMEM_SHARED((BR * 256 + 2048,), jnp.float32),
    ],
)


CHS = 8192
GB = 128


def _agg_body(kdst_hbm, kex_hbm, den_hbm, feat_hbm, rst_hbm,
              denloc, dentmp, acc, kdbuf, slotbuf, srcidx, srcidxa, srcidxb,
              dstl, exrow, albuf, featb0, featb1, sem):
    c = lax.axis_index("c")
    t = lax.axis_index("s")
    d0 = c * (N // 2) + t * 128

    pltpu.sync_copy(den_hbm.at[pl.ds(d0 * 4, 512)], denloc)
    pltpu.sync_copy(den_hbm.at[pl.ds(DEN + d0 * 4, 512)], dentmp)

    def dsum(i, _):
        denloc[pl.ds(i * 16, 16)] = (denloc[pl.ds(i * 16, 16)]
                                     + dentmp[pl.ds(i * 16, 16)])
        return 0
    lax.fori_loop(0, 512 // 16, dsum, 0, unroll=8)

    def za(i, _):
        acc[pl.ds(i * 16, 16)] = jnp.zeros((16,), jnp.float32)
        return 0
    lax.fori_loop(0, 128 * 256 // 16, za, 0, unroll=8)

    def chunk_body(ch, _):
        pltpu.sync_copy(kdst_hbm.at[pl.ds(ch * CHS, CHS)], kdbuf)

        def scan(i, cur):
            dv = kdbuf[pl.ds(i * 16, 16)]
            m = (dv >= d0) & (dv < d0 + 128)
            plsc.store_compressed(slotbuf.at[pl.ds(cur, 16)],
                                  _iota16() + i * 16, mask=m)
            return cur + jnp.max(plsc.all_reduce_population_count(m))

        nm = lax.fori_loop(0, CHS // 16, scan, 0, unroll=4)
        for k in range(8):
            slotbuf[pl.ds(nm + k * 16, 16)] = jnp.zeros((16,), jnp.int32)
        ng = lax.div(nm + GB - 1, GB)

        def group_body(g, _):
            lanebase = g * GB
            for i in range(GB // 16):
                slot = slotbuf[pl.ds(lanebase + i * 16, 16)]
                live = (_iota16() + lanebase + i * 16) < nm
                slot = jnp.where(live, slot, 0)
                sidx = (lax.shift_right_logical(slot, 5)
                        + ch * (CHS // NSLOT))
                srcidx[pl.ds(i * 16, 16)] = sidx
                srcidxa[pl.ds(i * 16, 16)] = sidx * 2
                srcidxb[pl.ds(i * 16, 16)] = sidx * 2 + 1
                dl = plsc.load_gather(kdbuf, [slot]) - d0
                dstl[pl.ds(i * 16, 16)] = jnp.where(live, dl, 0)

            cp1 = pltpu.async_copy(kex_hbm.at[srcidx.at[pl.ds(0, GB)]], exrow, sem)
            cp2 = pltpu.async_copy(feat_hbm.at[srcidxa.at[pl.ds(0, GB)]], featb0, sem)
            cp3 = pltpu.async_copy(feat_hbm.at[srcidxb.at[pl.ds(0, GB)]], featb1, sem)
            cp1.wait()

            for i in range(GB // 16):
                slot = slotbuf[pl.ds(lanebase + i * 16, 16)]
                live = (_iota16() + lanebase + i * 16) < nm
                slot = jnp.where(live, slot, 0)
                kk = lax.bitwise_and(slot, NSLOT - 1)
                dl = dstl[pl.ds(i * 16, 16)]
                erow = _iota16() + i * 16
                for hh in range(HEADS):
                    ex = plsc.load_gather(exrow, [erow, hh * 32 + kk])
                    dv = plsc.load_gather(denloc, [dl * 4 + hh])
                    al = jnp.where(live & (ex > 0), ex / dv, 0.0)
                    albuf[pl.ds(hh * 128 + i * 16, 16)] = al

            cp2.wait()
            cp3.wait()

            def edge_body(e, _):
                db = jnp.max(plsc.load_gather(dstl, [_splat_i(e)]))
                ah = [plsc.load_gather(albuf, [_splat_i(hh * 128 + e)])
                      for hh in range(HEADS)]
                for dd in range(16):
                    a = ah[dd // 4]
                    if dd < 8:
                        f = featb0[e, pl.ds(dd * 16, 16)]
                    else:
                        f = featb1[e, pl.ds(dd * 16 - 128, 16)]
                    o = db * 256 + dd * 16
                    acc[pl.ds(o, 16)] = acc[pl.ds(o, 16)] + f * a
                return 0

            lax.fori_loop(0, GB, edge_body, 0)
            return 0

        lax.fori_loop(0, ng, group_body, 0)
        return 0

    lax.fori_loop(0, N * NSLOT // CHS, chunk_body, 0)
    pltpu.sync_copy(acc, rst_hbm.at[pl.ds(d0 * 256, 128 * 256)])


_agg_call = pl.kernel(
    _agg_body,
    out_type=jax.ShapeDtypeStruct((N * 256,), jnp.float32),
    mesh=_mesh(),
    compiler_params=_params,
    scratch_types=[
        pltpu.VMEM((512,), jnp.float32),
        pltpu.VMEM((512,), jnp.float32),
        pltpu.VMEM((128 * 256,), jnp.float32),
        pltpu.VMEM((CHS,), jnp.int32),
        pltpu.VMEM((CHS + 256,), jnp.int32),
        pltpu.VMEM((GB,), jnp.int32),
        pltpu.VMEM((GB,), jnp.int32),
        pltpu.VMEM((GB,), jnp.int32),
        pltpu.VMEM((GB,), jnp.int32),
        pltpu.VMEM((GB, 128), jnp.float32),
        pltpu.VMEM((HEADS * GB,), jnp.float32),
        pltpu.VMEM((GB, 128), jnp.float32),
        pltpu.VMEM((GB, 128), jnp.float32),
        pltpu.SemaphoreType.DMA,
    ],
)


def kernel(edge_index, h, simlar, W_trans, W_gat, attn_l, attn_r, b_gat,
           Ws1, bs1, Ws2):
    src, dst = edge_index[0], edge_index[1]

    feat, el, er_t = pl.pallas_call(
        _feat_kernel,
        out_shape=(
            jax.ShapeDtypeStruct((N, D), jnp.float32),
            jax.ShapeDtypeStruct((N, HEADS), jnp.float32),
            jax.ShapeDtypeStruct((HEADS, N), jnp.float32),
        ),
    )(h, W_trans, W_gat, attn_l, attn_r)

    sval = _sval_call(src, dst, simlar.reshape(N * N // 128, 128))
    eflat = src * N + dst
    kdst, kex, den = _graph_call(eflat, sval, el.reshape(-1), er_t)
    rst = _agg_call(kdst, kex.reshape(N, 128), den,
                    feat.reshape(2 * N, 128))

    out = pl.pallas_call(
        _final_kernel,
        out_shape=jax.ShapeDtypeStruct((N, D), jnp.float32),
    )(rst.reshape(N, D), jnp.broadcast_to(b_gat[None], (N, D)))
    return out


# speedup vs baseline: 1.0073x; 1.0073x over previous
import jax
import jax.numpy as jnp
from jax import lax
from jax.experimental import pallas as pl
from jax.experimental.pallas import tpu as pltpu
from jax.experimental.pallas import tpu_sc as plsc

N = 4096
E = 131072
IN = 128
HID = 64
HEADS = 4
D = HID * HEADS
T = 29

NW = 32
EPW = E // NW
EPT = E // 16
BR = 128
NB = N // 2 // BR
DEADCOL = N
DEN = 4 * N + 2048
NSLOT = 32


def _mesh():
    return plsc.VectorSubcoreMesh(core_axis_name="c", subcore_axis_name="s")


_params = pltpu.CompilerParams(needs_layout_passes=False)


def _iota16():
    return lax.iota(jnp.int32, 16)


def _splat_i(x):
    return jnp.full((16,), x, jnp.int32)


def _feat_kernel(h_ref, wt_ref, wg_ref, al_ref, ar_ref, feat_ref, el_ref, ert_ref):
    hw = lax.dot_general(h_ref[...], wt_ref[...], (((1,), (1,)), ((), ())),
                         precision=lax.Precision.HIGHEST)
    hh = jnp.maximum(hw, 0.01 * hw)
    feat = lax.dot_general(hh, wg_ref[...], (((1,), (1,)), ((), ())),
                           precision=lax.Precision.HIGHEST)
    feat_ref[...] = feat
    fh = feat.reshape(N, HEADS, HID)
    el_ref[...] = jnp.sum(fh * al_ref[...][None], axis=-1)
    ert_ref[...] = jnp.sum(fh * ar_ref[...][None], axis=-1).T


def _final_kernel(rst_ref, b_ref, out_ref):
    x = rst_ref[...] + b_ref[...]
    out_ref[...] = jnp.where(x > 0, x, jnp.exp(jnp.minimum(x, 0.0)) - 1.0)


CH = 128


def _sval_body(src_hbm, dst_hbm, sim_hbm, out_hbm, rowi, lanes, svalv,
               rowbuf0, rowbuf1, sem):
    c = lax.axis_index("c")
    s = lax.axis_index("s")
    wid = s * 2 + c
    base = wid * EPW
    pltpu.sync_copy(src_hbm.at[pl.ds(base, EPW)], rowi)
    pltpu.sync_copy(dst_hbm.at[pl.ds(base, EPW)], lanes)

    def fill(i, _):
        sv = rowi[pl.ds(i * 16, 16)]
        dv = lanes[pl.ds(i * 16, 16)]
        rowi[pl.ds(i * 16, 16)] = sv * 32 + lax.shift_right_logical(dv, 7)
        lanes[pl.ds(i * 16, 16)] = lax.bitwise_and(dv, 127)
        return 0

    lax.fori_loop(0, EPW // 16, fill, 0, unroll=4)

    nch = EPW // CH
    bufs = (rowbuf0, rowbuf1)
    started = [
        pltpu.async_copy(sim_hbm.at[rowi.at[pl.ds(j * CH, CH)]], bufs[j], sem)
        for j in range(2)
    ]

    for j in range(nch):
        started[j].wait()

        def extract(i, _):
            p = _iota16() + i * 16
            lane = lanes[pl.ds(j * CH + i * 16, 16)]
            svalv[pl.ds(j * CH + i * 16, 16)] = plsc.load_gather(
                bufs[j % 2], [p, lane])
            return 0

        lax.fori_loop(0, CH // 16, extract, 0, unroll=4)
        if j + 2 < nch:
            started.append(
                pltpu.async_copy(
                    sim_hbm.at[rowi.at[pl.ds((j + 2) * CH, CH)]],
                    bufs[j % 2], sem))

    pltpu.sync_copy(svalv, out_hbm.at[pl.ds(base, EPW)])


_sval_call = pl.kernel(
    _sval_body,
    out_type=jax.ShapeDtypeStruct((E,), jnp.float32),
    mesh=_mesh(),
    compiler_params=_params,
    scratch_types=[
        pltpu.VMEM((EPW,), jnp.int32),
        pltpu.VMEM((EPW,), jnp.int32),
        pltpu.VMEM((EPW,), jnp.float32),
        pltpu.VMEM((CH, 128), jnp.float32),
        pltpu.VMEM((CH, 128), jnp.float32),
        pltpu.SemaphoreType.DMA,
    ],
)


def _graph_body(ef_hbm, sv_hbm, el_hbm, ert_hbm,
                kdst_hbm, kex_hbm, den_hbm,
                eflat, svv, fidx, fval, tmp128, tmp128m, erbuf, elbuf,
                rowbuf, markbuf, gbuf, onesb, kcol, ccol, kdst_st, kex_st,
                denidx, denval, zbuf, acc, den_sh, mark_sh):
    c = lax.axis_index("c")
    t = lax.axis_index("s")

    ebase = t * EPT
    pltpu.sync_copy(ef_hbm.at[pl.ds(ebase, EPT)], eflat)
    pltpu.sync_copy(sv_hbm.at[pl.ds(ebase, EPT)], svv)
    pltpu.sync_copy(ert_hbm, erbuf)

    def zb(i, _):
        zbuf[pl.ds(i * 16, 16)] = jnp.zeros((16,), jnp.float32)
        return 0
    lax.fori_loop(0, 8192 // 16, zb, 0, unroll=8)

    def ob(i, _):
        onesb[pl.ds(i * 16, 16)] = jnp.ones((16,), jnp.float32)
        return 0
    lax.fori_loop(0, 128 // 16, ob, 0)

    @pl.when(t == 0)
    def _():
        pltpu.sync_copy(zbuf, den_sh.at[pl.ds(0, 8192)])
        pltpu.sync_copy(zbuf, den_sh.at[pl.ds(8192, 8192)])
        pltpu.sync_copy(zbuf.at[pl.ds(0, 2048)], den_sh.at[pl.ds(16384, 2048)])

    def block_body(b, _):
        blo = c * (N // 2) + b * BR

        plsc.subcore_barrier()
        for k in range(4):
            pltpu.sync_copy(zbuf, acc.at[pl.ds(t * (BR * N // 16) + k * 8192,
                                               8192)])
        pltpu.sync_copy(zbuf.at[pl.ds(0, 2048)],
                        mark_sh.at[pl.ds(t * 2048, 2048)])

        @pl.when(t == 0)
        def _():
            pltpu.sync_copy(zbuf.at[pl.ds(0, 2048)],
                            mark_sh.at[pl.ds(BR * 256, 2048)])
        plsc.subcore_barrier()

        def scan(i, cnt):
            ef = eflat[pl.ds(i * 16, 16)]
            sv = lax.shift_right_logical(ef, 12)
            vv = svv[pl.ds(i * 16, 16)]
            inb = (sv >= blo) & (sv < blo + BR)
            lidx = ef - blo * N
            plsc.store_compressed(fidx.at[pl.ds(cnt, 16)], lidx, mask=inb)
            plsc.store_compressed(fval.at[pl.ds(cnt, 16)], vv, mask=inb)
            n = jnp.max(plsc.all_reduce_population_count(inb))
            return cnt + n

        cnt = lax.fori_loop(0, EPT // 16, scan, 0)
        for k in range(8):
            fidx[pl.ds(cnt + k * 16, 16)] = _splat_i(BR * N)
            fval[pl.ds(cnt + k * 16, 16)] = jnp.zeros((16,), jnp.float32)
        nst = lax.div(cnt + 127, 128)

        def stream_j(j, _):
            for k in range(8):
                fv = fidx[pl.ds(j * 128 + k * 16, 16)]
                tmp128[pl.ds(k * 16, 16)] = fv
                tmp128m[pl.ds(k * 16, 16)] = lax.shift_right_logical(fv, 4)
            pltpu.sync_copy(fval.at[pl.ds(j * 128, 128)], acc.at[tmp128],
                            add=True)
            pltpu.sync_copy(onesb, mark_sh.at[tmp128m], add=True)
            return 0

        lax.fori_loop(0, nst, stream_j, 0)
        plsc.subcore_barrier()

        rpt = BR // 16
        r0 = blo + t * rpt
        pltpu.sync_copy(el_hbm.at[pl.ds(r0 * 4, rpt * 4)], elbuf)

        def row_body(rr, _):
            pltpu.sync_copy(acc.at[pl.ds((t * rpt + rr) * N, N)], rowbuf)
            pltpu.sync_copy(mark_sh.at[pl.ds((t * rpt + rr) * 256, 256)],
                            markbuf)

            def mscan(j, cur):
                m = markbuf[pl.ds(j * 16, 16)] > 0.0
                plsc.store_compressed(gbuf.at[pl.ds(cur, 16)],
                                      _iota16() + j * 16, mask=m)
                return cur + jnp.max(plsc.all_reduce_population_count(m))

            ngrp = lax.fori_loop(0, 16, mscan, 0)

            def comp(gi, cur):
                g = jnp.max(plsc.load_gather(gbuf, [_splat_i(gi)]))
                v = rowbuf[pl.ds(g * 16, 16)]
                m = v > 0.0
                plsc.store_compressed(ccol.at[pl.ds(cur, 16)],
                                      _iota16() + g * 16, mask=m)
                return cur + jnp.max(plsc.all_reduce_population_count(m))

            P = lax.fori_loop(0, ngrp, comp, 0)
            kc = jnp.minimum(P, T)

            def take_all(_):
                kcol[pl.ds(0, 16)] = ccol[pl.ds(0, 16)]
                kcol[pl.ds(16, 16)] = ccol[pl.ds(16, 16)]
                return 0

            def select29(_):
                def pick(k, _):
                    def mx_scan(j, mx):
                        return jnp.maximum(mx, rowbuf[pl.ds(j * 16, 16)])

                    mxv = lax.fori_loop(0, N // 16, mx_scan,
                                        jnp.full((16,), -jnp.inf, jnp.float32))
                    mxs = jnp.max(mxv)

                    def find(j, pos):
                        v = rowbuf[pl.ds(j * 16, 16)]
                        m = v == mxs
                        anyc = jnp.max(
                            plsc.all_reduce_population_count(m)) > 0
                        ffs = jnp.max(plsc.all_reduce_ffs(m))
                        return jnp.where((pos < 0) & anyc, j * 16 + ffs, pos)

                    pos = lax.fori_loop(0, N // 16, find, jnp.int32(-1))
                    plsc.store_scatter(kcol, [_splat_i(k)], _splat_i(pos),
                                       mask=_iota16() == 0)
                    plsc.store_scatter(rowbuf, [_splat_i(pos)],
                                       jnp.full((16,), -jnp.inf, jnp.float32),
                                       mask=_iota16() == 0)
                    return 0

                lax.fori_loop(0, T, pick, 0)
                return 0

            lax.cond(P <= T, take_all, select29, 0)

            for half in range(2):
                colv = kcol[pl.ds(half * 16, 16)]
                vmask = (_iota16() + half * 16) < kc
                colk = jnp.where(vmask, colv, DEADCOL)
                col_er = jnp.where(vmask, colv, 0)
                kdst_st[pl.ds(rr * NSLOT + half * 16, 16)] = colk
                for hh in range(HEADS):
                    erh = plsc.load_gather(erbuf, [_splat_i(hh), col_er])
                    els = plsc.load_gather(elbuf, [_splat_i(rr * 4 + hh)])
                    e = els + erh
                    e = jnp.where(e > 0, e, 0.2 * e)
                    ex = jnp.where(vmask, jnp.exp(e), 0.0)
                    off = rr * 128 + hh * 32 + half * 16
                    kex_st[pl.ds(off, 16)] = ex
                    denidx[pl.ds(off, 16)] = colk * 4 + hh
                    denval[pl.ds(off, 16)] = ex
            return 0

        lax.fori_loop(0, rpt, row_body, 0)

        def den_j(j, _):
            for k in range(8):
                tmp128[pl.ds(k * 16, 16)] = denidx[pl.ds(j * 128 + k * 16, 16)]
            pltpu.sync_copy(denval.at[pl.ds(j * 128, 128)], den_sh.at[tmp128],
                            add=True)
            return 0

        lax.fori_loop(0, rpt, den_j, 0)

        pltpu.sync_copy(kdst_st, kdst_hbm.at[pl.ds(r0 * NSLOT, rpt * NSLOT)])
        pltpu.sync_copy(kex_st, kex_hbm.at[pl.ds(r0 * 128, rpt * 128)])
        return 0

    lax.fori_loop(0, NB, block_body, 0)
    plsc.subcore_barrier()

    @pl.when(t == 0)
    def _():
        pltpu.sync_copy(den_sh, den_hbm.at[pl.ds(c * DEN, DEN)])


_graph_call = pl.kernel(
    _graph_body,
    out_type=(
        jax.ShapeDtypeStruct((N * NSLOT,), jnp.int32),
        jax.ShapeDtypeStruct((N * 128,), jnp.float32),
        jax.ShapeDtypeStruct((2 * DEN,), jnp.float32),
    ),
    mesh=_mesh(),
    compiler_params=_params,
    scratch_types=[
        pltpu.VMEM((EPT,), jnp.int32),
        pltpu.VMEM((EPT,), jnp.float32),
        pltpu.VMEM((EPT + 256,), jnp.int32),
        pltpu.VMEM((EPT + 256,), jnp.float32),
        pltpu.VMEM((128,), jnp.int32),
        pltpu.VMEM((128,), jnp.int32),
        pltpu.VMEM((HEADS, N), jnp.float32),
        pltpu.VMEM((32,), jnp.float32),
        pltpu.VMEM((N,), jnp.float32),
        pltpu.VMEM((256,), jnp.float32),
        pltpu.VMEM((272,), jnp.int32),
        pltpu.VMEM((128,), jnp.float32),
        pltpu.VMEM((48,), jnp.int32),
        pltpu.VMEM((N + 16,), jnp.int32),
        pltpu.VMEM((BR // 16 * NSLOT,), jnp.int32),
        pltpu.VMEM((BR // 16 * 128,), jnp.float32),
        pltpu.VMEM((BR // 16 * 128,), jnp.int32),
        pltpu.VMEM((BR // 16 * 128,), jnp.float32),
        pltpu.VMEM((8192,), jnp.float32),
        pltpu.VMEM_SHARED((BR * N + 2048,), jnp.float32),
        pltpu.VMEM_SHARED((DEN,), jnp.float32),
        pltpu.VMEM_SHARED((BR * 256 + 2048,), jnp.float32),
    ],
)


CHS = 8192
GB = 128


def _agg_body(kdst_hbm, kex_hbm, den_hbm, feat_hbm, rst_hbm,
              denloc, dentmp, acc, kdbuf, slotbuf, srcidx, srcidxa, srcidxb,
              dstl, exrow, albuf, featb0, featb1, sem):
    c = lax.axis_index("c")
    t = lax.axis_index("s")
    d0 = c * (N // 2) + t * 128

    pltpu.sync_copy(den_hbm.at[pl.ds(d0 * 4, 512)], denloc)
    pltpu.sync_copy(den_hbm.at[pl.ds(DEN + d0 * 4, 512)], dentmp)

    def dsum(i, _):
        denloc[pl.ds(i * 16, 16)] = (denloc[pl.ds(i * 16, 16)]
                                     + dentmp[pl.ds(i * 16, 16)])
        return 0
    lax.fori_loop(0, 512 // 16, dsum, 0, unroll=8)

    def za(i, _):
        acc[pl.ds(i * 16, 16)] = jnp.zeros((16,), jnp.float32)
        return 0
    lax.fori_loop(0, 128 * 256 // 16, za, 0, unroll=8)

    def chunk_body(ch, _):
        pltpu.sync_copy(kdst_hbm.at[pl.ds(ch * CHS, CHS)], kdbuf)

        def scan(i, cur):
            dv = kdbuf[pl.ds(i * 16, 16)]
            m = (dv >= d0) & (dv < d0 + 128)
            plsc.store_compressed(slotbuf.at[pl.ds(cur, 16)],
                                  _iota16() + i * 16, mask=m)
            return cur + jnp.max(plsc.all_reduce_population_count(m))

        nm = lax.fori_loop(0, CHS // 16, scan, 0)
        for k in range(8):
            slotbuf[pl.ds(nm + k * 16, 16)] = jnp.zeros((16,), jnp.int32)
        ng = lax.div(nm + GB - 1, GB)

        def group_body(g, _):
            lanebase = g * GB
            for i in range(GB // 16):
                slot = slotbuf[pl.ds(lanebase + i * 16, 16)]
                live = (_iota16() + lanebase + i * 16) < nm
                slot = jnp.where(live, slot, 0)
                sidx = (lax.shift_right_logical(slot, 5)
                        + ch * (CHS // NSLOT))
                srcidx[pl.ds(i * 16, 16)] = sidx
                srcidxa[pl.ds(i * 16, 16)] = sidx * 2
                srcidxb[pl.ds(i * 16, 16)] = sidx * 2 + 1
                dl = plsc.load_gather(kdbuf, [slot]) - d0
                dstl[pl.ds(i * 16, 16)] = jnp.where(live, dl, 0)

            cp1 = pltpu.async_copy(kex_hbm.at[srcidx.at[pl.ds(0, GB)]], exrow, sem)
            cp2 = pltpu.async_copy(feat_hbm.at[srcidxa.at[pl.ds(0, GB)]], featb0, sem)
            cp3 = pltpu.async_copy(feat_hbm.at[srcidxb.at[pl.ds(0, GB)]], featb1, sem)
            cp1.wait()

            for i in range(GB // 16):
                slot = slotbuf[pl.ds(lanebase + i * 16, 16)]
                live = (_iota16() + lanebase + i * 16) < nm
                slot = jnp.where(live, slot, 0)
                kk = lax.bitwise_and(slot, NSLOT - 1)
                dl = dstl[pl.ds(i * 16, 16)]
                erow = _iota16() + i * 16
                for hh in range(HEADS):
                    ex = plsc.load_gather(exrow, [erow, hh * 32 + kk])
                    dv = plsc.load_gather(denloc, [dl * 4 + hh])
                    al = jnp.where(live & (ex > 0), ex / dv, 0.0)
                    albuf[pl.ds(hh * 128 + i * 16, 16)] = al

            cp2.wait()
            cp3.wait()

            def edge_body(e, _):
                db = jnp.max(plsc.load_gather(dstl, [_splat_i(e)]))
                ah = [plsc.load_gather(albuf, [_splat_i(hh * 128 + e)])
                      for hh in range(HEADS)]
                for dd in range(16):
                    a = ah[dd // 4]
                    if dd < 8:
                        f = featb0[e, pl.ds(dd * 16, 16)]
                    else:
                        f = featb1[e, pl.ds(dd * 16 - 128, 16)]
                    o = db * 256 + dd * 16
                    acc[pl.ds(o, 16)] = acc[pl.ds(o, 16)] + f * a
                return 0

            lax.fori_loop(0, GB, edge_body, 0)
            return 0

        lax.fori_loop(0, ng, group_body, 0)
        return 0

    lax.fori_loop(0, N * NSLOT // CHS, chunk_body, 0)
    pltpu.sync_copy(acc, rst_hbm.at[pl.ds(d0 * 256, 128 * 256)])


_agg_call = pl.kernel(
    _agg_body,
    out_type=jax.ShapeDtypeStruct((N * 256,), jnp.float32),
    mesh=_mesh(),
    compiler_params=_params,
    scratch_types=[
        pltpu.VMEM((512,), jnp.float32),
        pltpu.VMEM((512,), jnp.float32),
        pltpu.VMEM((128 * 256,), jnp.float32),
        pltpu.VMEM((CHS,), jnp.int32),
        pltpu.VMEM((CHS + 256,), jnp.int32),
        pltpu.VMEM((GB,), jnp.int32),
        pltpu.VMEM((GB,), jnp.int32),
        pltpu.VMEM((GB,), jnp.int32),
        pltpu.VMEM((GB,), jnp.int32),
        pltpu.VMEM((GB, 128), jnp.float32),
        pltpu.VMEM((HEADS * GB,), jnp.float32),
        pltpu.VMEM((GB, 128), jnp.float32),
        pltpu.VMEM((GB, 128), jnp.float32),
        pltpu.SemaphoreType.DMA,
    ],
)


def kernel(edge_index, h, simlar, W_trans, W_gat, attn_l, attn_r, b_gat,
           Ws1, bs1, Ws2):
    src, dst = edge_index[0], edge_index[1]

    feat, el, er_t = pl.pallas_call(
        _feat_kernel,
        out_shape=(
            jax.ShapeDtypeStruct((N, D), jnp.float32),
            jax.ShapeDtypeStruct((N, HEADS), jnp.float32),
            jax.ShapeDtypeStruct((HEADS, N), jnp.float32),
        ),
    )(h, W_trans, W_gat, attn_l, attn_r)

    sval = _sval_call(src, dst, simlar.reshape(N * N // 128, 128))
    eflat = src * N + dst
    kdst, kex, den = _graph_call(eflat, sval, el.reshape(-1), er_t)
    rst = _agg_call(kdst, kex.reshape(N, 128), den,
                    feat.reshape(2 * N, 128))

    out = pl.pallas_call(
        _final_kernel,
        out_shape=jax.ShapeDtypeStruct((N, D), jnp.float32),
    )(rst.reshape(N, D), jnp.broadcast_to(b_gat[None], (N, D)))
    return out
